# Initial kernel scaffold; baseline (speedup 1.0000x reference)
import functools

import jax
import jax.numpy as jnp
from jax import lax
from jax.experimental import pallas as pl
from jax.experimental.pallas import tpu as pltpu
from jax.experimental.pallas import tpu_sc as plsc

NUM_CORES = 2
NUM_SUBCORES = 16
LANES = 16
NUM_WORKERS = NUM_CORES * NUM_SUBCORES

TOTAL = 4096 * 8192
PER_WORKER = TOTAL // NUM_WORKERS
CHUNK = 16384
NUM_CHUNKS = PER_WORKER // CHUNK


def _make_kernel():
    mesh = plsc.VectorSubcoreMesh(core_axis_name="c", subcore_axis_name="s")

    @functools.partial(
        pl.kernel,
        out_type=jax.ShapeDtypeStruct((TOTAL,), jnp.int32),
        mesh=mesh,
        scratch_types=[
            pltpu.VMEM((CHUNK,), jnp.float32),
            pltpu.VMEM((CHUNK,), jnp.int32),
            pltpu.VMEM((72,), jnp.float32),
            pltpu.VMEM((16,), jnp.float32),
            pltpu.VMEM((16,), jnp.float32),
        ],
    )
    def bucketize(x_hbm, bpad_hbm, b0_hbm, inv_hbm, out_hbm,
                  in_v, out_v, bpad_v, b0_v, inv_v):
        wid = lax.axis_index("s") * NUM_CORES + lax.axis_index("c")
        base = wid * PER_WORKER

        pltpu.sync_copy(bpad_hbm, bpad_v)
        pltpu.sync_copy(b0_hbm, b0_v)
        pltpu.sync_copy(inv_hbm, inv_v)

        b0 = b0_v[...]
        inv = inv_v[...]

        def do_chunk(ch, carry):
            off = base + ch * CHUNK
            pltpu.sync_copy(x_hbm.at[pl.ds(off, CHUNK)], in_v)

            def step(i, c):
                x = in_v[pl.ds(i * LANES, LANES)]
                t = (x - b0) * inv
                t = jnp.minimum(jnp.maximum(t, -1.0), 62.5)
                g = t.astype(jnp.int32) + 1
                g1 = g + 1
                lo = plsc.load_gather(bpad_v, [g])
                hi = plsc.load_gather(bpad_v, [g1])
                idx = jnp.where(x >= hi, g1, g)
                idx = jnp.where(x < lo, idx - 1, idx)
                out_v[pl.ds(i * LANES, LANES)] = idx
                return c

            lax.fori_loop(0, CHUNK // LANES, step, 0)
            pltpu.sync_copy(out_v, out_hbm.at[pl.ds(off, CHUNK)])
            return carry

        lax.fori_loop(0, NUM_CHUNKS, do_chunk, 0)

    return bucketize


_BUCKETIZE = _make_kernel()


@jax.jit
def kernel(inputs, bin_boundaries):
    b = bin_boundaries.astype(jnp.float32)
    b0 = b[0]
    inv = 62.0 / (b[62] - b[0])
    bpad = (
        jnp.full((72,), jnp.inf, jnp.float32)
        .at[0]
        .set(-jnp.inf)
        .at[1:64]
        .set(b)
    )
    b0_arr = jnp.full((16,), b0, jnp.float32)
    inv_arr = jnp.full((16,), inv, jnp.float32)
    x = inputs.reshape(-1)
    out = _BUCKETIZE(x, bpad, b0_arr, inv_arr)
    return out.reshape(inputs.shape)


# speedup vs baseline: 6.8357x; 6.8357x over previous
import functools

import jax
import jax.numpy as jnp
from jax import lax
from jax.experimental import pallas as pl
from jax.experimental.pallas import tpu as pltpu
from jax.experimental.pallas import tpu_sc as plsc

NUM_CORES = 2
NUM_SUBCORES = 16
LANES = 16
NUM_WORKERS = NUM_CORES * NUM_SUBCORES

TOTAL = 4096 * 8192
PER_WORKER = TOTAL // NUM_WORKERS
CHUNK = 16384
NUM_CHUNKS = PER_WORKER // CHUNK


def _make_kernel():
    mesh = plsc.VectorSubcoreMesh(core_axis_name="c", subcore_axis_name="s")

    @functools.partial(
        pl.kernel,
        out_type=jax.ShapeDtypeStruct((TOTAL,), jnp.int32),
        mesh=mesh,
        compiler_params=pltpu.CompilerParams(needs_layout_passes=False),
        scratch_types=[
            pltpu.VMEM((CHUNK,), jnp.float32),
            pltpu.VMEM((CHUNK,), jnp.int32),
            pltpu.VMEM((72,), jnp.float32),
            pltpu.VMEM((16,), jnp.float32),
            pltpu.VMEM((16,), jnp.float32),
        ],
    )
    def bucketize(x_hbm, bpad_hbm, b0_hbm, inv_hbm, out_hbm,
                  in_v, out_v, bpad_v, b0_v, inv_v):
        wid = lax.axis_index("s") * NUM_CORES + lax.axis_index("c")
        base = wid * PER_WORKER

        pltpu.sync_copy(bpad_hbm, bpad_v)
        pltpu.sync_copy(b0_hbm, b0_v)
        pltpu.sync_copy(inv_hbm, inv_v)

        b0 = b0_v[...]
        inv = inv_v[...]

        def do_chunk(ch, carry):
            off = base + ch * CHUNK
            pltpu.sync_copy(x_hbm.at[pl.ds(off, CHUNK)], in_v)

            def step(i, c):
                x = in_v[pl.ds(i * LANES, LANES)]
                t = (x - b0) * inv
                t = jnp.minimum(jnp.maximum(t, -1.0), 62.5)
                g = t.astype(jnp.int32) + 1
                g1 = g + 1
                lo = plsc.load_gather(bpad_v, [g])
                hi = plsc.load_gather(bpad_v, [g1])
                idx = jnp.where(x >= hi, g1, g)
                idx = jnp.where(x < lo, idx - 1, idx)
                out_v[pl.ds(i * LANES, LANES)] = idx
                return c

            lax.fori_loop(0, CHUNK // LANES, step, 0)
            pltpu.sync_copy(out_v, out_hbm.at[pl.ds(off, CHUNK)])
            return carry

        lax.fori_loop(0, NUM_CHUNKS, do_chunk, 0)

    return bucketize


_BUCKETIZE = _make_kernel()


@jax.jit
def kernel(inputs, bin_boundaries):
    b = bin_boundaries.astype(jnp.float32)
    b0 = b[0]
    inv = 62.0 / (b[62] - b[0])
    bpad = (
        jnp.full((72,), jnp.inf, jnp.float32)
        .at[0]
        .set(-jnp.inf)
        .at[1:64]
        .set(b)
    )
    b0_arr = jnp.full((16,), b0, jnp.float32)
    inv_arr = jnp.full((16,), inv, jnp.float32)
    x = inputs.reshape(-1)
    out = _BUCKETIZE(x, bpad, b0_arr, inv_arr)
    return out.reshape(inputs.shape)


# speedup vs baseline: 11.6976x; 1.7113x over previous
import functools

import jax
import jax.numpy as jnp
from jax import lax
from jax.experimental import pallas as pl
from jax.experimental.pallas import tpu as pltpu
from jax.experimental.pallas import tpu_sc as plsc

NUM_CORES = 2
NUM_SUBCORES = 16
LANES = 16
NUM_WORKERS = NUM_CORES * NUM_SUBCORES

TOTAL = 4096 * 8192
PER_WORKER = TOTAL // NUM_WORKERS
CHUNK = 16384
NUM_CHUNKS = PER_WORKER // CHUNK


def _make_kernel():
    mesh = plsc.VectorSubcoreMesh(core_axis_name="c", subcore_axis_name="s")

    @functools.partial(
        pl.kernel,
        out_type=jax.ShapeDtypeStruct((TOTAL,), jnp.int32),
        mesh=mesh,
        compiler_params=pltpu.CompilerParams(needs_layout_passes=False),
        scratch_types=[
            pltpu.VMEM((CHUNK,), jnp.float32),
            pltpu.VMEM((CHUNK,), jnp.int32),
            pltpu.VMEM((72,), jnp.float32),
            pltpu.VMEM((16,), jnp.float32),
            pltpu.VMEM((16,), jnp.float32),
        ],
    )
    def bucketize(x_hbm, bpad_hbm, b0_hbm, inv_hbm, out_hbm,
                  in_v, out_v, bpad_v, b0_v, inv_v):
        wid = lax.axis_index("s") * NUM_CORES + lax.axis_index("c")
        base = wid * PER_WORKER

        pltpu.sync_copy(bpad_hbm, bpad_v)
        pltpu.sync_copy(b0_hbm, b0_v)
        pltpu.sync_copy(inv_hbm, inv_v)

        b0 = b0_v[...]
        inv = inv_v[...]

        def do_chunk(ch, carry):
            off = base + ch * CHUNK
            pltpu.sync_copy(x_hbm.at[pl.ds(off, CHUNK)], in_v)

            @plsc.parallel_loop(0, CHUNK, step=LANES, unroll=8)
            def step(i):
                x = in_v[pl.ds(i, LANES)]
                t = (x - b0) * inv
                t = jnp.minimum(jnp.maximum(t, -1.0), 63.5)
                tr = t.astype(jnp.int32)
                g = tr + 1
                lo = plsc.load_gather(bpad_v, [g])
                out_v[pl.ds(i, LANES)] = jnp.where(x < lo, tr, g)

            pltpu.sync_copy(out_v, out_hbm.at[pl.ds(off, CHUNK)])
            return carry

        lax.fori_loop(0, NUM_CHUNKS, do_chunk, 0)

    return bucketize


_BUCKETIZE = _make_kernel()


@jax.jit
def kernel(inputs, bin_boundaries):
    b = bin_boundaries.astype(jnp.float32)
    b0 = b[0]
    inv = (62.0 / (b[62] - b[0])) * jnp.float32(1.0 + 1e-5)
    bpad = (
        jnp.full((72,), jnp.inf, jnp.float32)
        .at[0]
        .set(-jnp.inf)
        .at[1:64]
        .set(b)
    )
    b0_arr = jnp.full((16,), b0, jnp.float32)
    inv_arr = jnp.full((16,), inv, jnp.float32)
    x = inputs.reshape(-1)
    out = _BUCKETIZE(x, bpad, b0_arr, inv_arr)
    return out.reshape(inputs.shape)


# speedup vs baseline: 15.6311x; 1.3363x over previous
import functools

import jax
import jax.numpy as jnp
from jax import lax
from jax.experimental import pallas as pl
from jax.experimental.pallas import tpu as pltpu
from jax.experimental.pallas import tpu_sc as plsc

NUM_CORES = 2
NUM_SUBCORES = 16
LANES = 16
NUM_WORKERS = NUM_CORES * NUM_SUBCORES

TOTAL = 4096 * 8192
PER_WORKER = TOTAL // NUM_WORKERS
CHUNK = 16384
NUM_CHUNKS = PER_WORKER // CHUNK


def _make_kernel():
    mesh = plsc.VectorSubcoreMesh(core_axis_name="c", subcore_axis_name="s")

    @functools.partial(
        pl.kernel,
        out_type=jax.ShapeDtypeStruct((TOTAL,), jnp.int32),
        mesh=mesh,
        compiler_params=pltpu.CompilerParams(needs_layout_passes=False),
        scratch_types=[
            pltpu.VMEM((CHUNK,), jnp.float32),
            pltpu.VMEM((CHUNK,), jnp.float32),
            pltpu.VMEM((CHUNK,), jnp.int32),
            pltpu.VMEM((CHUNK,), jnp.int32),
            pltpu.VMEM((72,), jnp.float32),
            pltpu.VMEM((16,), jnp.float32),
            pltpu.VMEM((16,), jnp.float32),
            pltpu.SemaphoreType.DMA,
            pltpu.SemaphoreType.DMA,
            pltpu.SemaphoreType.DMA,
            pltpu.SemaphoreType.DMA,
        ],
    )
    def bucketize(x_hbm, bpad_hbm, b0_hbm, inv_hbm, out_hbm,
                  in_v0, in_v1, out_v0, out_v1, bpad_v, b0_v, inv_v,
                  sem_i0, sem_i1, sem_o0, sem_o1):
        wid = lax.axis_index("s") * NUM_CORES + lax.axis_index("c")
        base = wid * PER_WORKER

        pltpu.sync_copy(bpad_hbm, bpad_v)
        pltpu.sync_copy(b0_hbm, b0_v)
        pltpu.sync_copy(inv_hbm, inv_v)

        b0 = b0_v[...]
        inv = inv_v[...]

        def compute(in_v, out_v):
            @plsc.parallel_loop(0, CHUNK, step=LANES, unroll=8)
            def step(i):
                x = in_v[pl.ds(i, LANES)]
                t = (x - b0) * inv
                t = jnp.minimum(jnp.maximum(t, -1.0), 63.5)
                tr = t.astype(jnp.int32)
                g = tr + 1
                lo = plsc.load_gather(bpad_v, [g])
                out_v[pl.ds(i, LANES)] = jnp.where(x < lo, tr, g)

        bufs = ((in_v0, out_v0, sem_i0, sem_o0),
                (in_v1, out_v1, sem_i1, sem_o1))

        pltpu.async_copy(x_hbm.at[pl.ds(base, CHUNK)], in_v0, sem_i0)
        pltpu.async_copy(x_hbm.at[pl.ds(base + CHUNK, CHUNK)], in_v1, sem_i1)

        @pl.loop(0, NUM_CHUNKS, step=2)
        def pair(ch):
            for b, (in_v, out_v, sem_i, sem_o) in enumerate(bufs):
                off = base + (ch + b) * CHUNK
                pltpu.make_async_copy(
                    x_hbm.at[pl.ds(off, CHUNK)], in_v, sem_i
                ).wait()

                @pl.when(ch > 0)
                def _wait_out():
                    pltpu.make_async_copy(
                        out_v, out_hbm.at[pl.ds(off - 2 * CHUNK, CHUNK)], sem_o
                    ).wait()

                compute(in_v, out_v)
                pltpu.async_copy(out_v, out_hbm.at[pl.ds(off, CHUNK)], sem_o)

                @pl.when(ch + 2 < NUM_CHUNKS)
                def _prefetch():
                    pltpu.async_copy(
                        x_hbm.at[pl.ds(off + 2 * CHUNK, CHUNK)], in_v, sem_i
                    )

        for b, (in_v, out_v, sem_i, sem_o) in enumerate(bufs):
            off = base + (NUM_CHUNKS - 2 + b) * CHUNK
            pltpu.make_async_copy(
                out_v, out_hbm.at[pl.ds(off, CHUNK)], sem_o
            ).wait()

    return bucketize


_BUCKETIZE = _make_kernel()


@jax.jit
def kernel(inputs, bin_boundaries):
    b = bin_boundaries.astype(jnp.float32)
    b0 = b[0]
    inv = (62.0 / (b[62] - b[0])) * jnp.float32(1.0 + 1e-5)
    bpad = (
        jnp.full((72,), jnp.inf, jnp.float32)
        .at[0]
        .set(-jnp.inf)
        .at[1:64]
        .set(b)
    )
    b0_arr = jnp.full((16,), b0, jnp.float32)
    inv_arr = jnp.full((16,), inv, jnp.float32)
    x = inputs.reshape(-1)
    out = _BUCKETIZE(x, bpad, b0_arr, inv_arr)
    return out.reshape(inputs.shape)


# speedup vs baseline: 39.4151x; 2.5216x over previous
import functools

import jax
import jax.numpy as jnp
from jax import lax
from jax.experimental import pallas as pl
from jax.experimental.pallas import tpu as pltpu
from jax.experimental.pallas import tpu_sc as plsc

NUM_CORES = 2
NUM_SUBCORES = 16
LANES = 16
NUM_WORKERS = NUM_CORES * NUM_SUBCORES

ROWS = 4096
COLS = 8192
ROWS_PER_WORKER = ROWS // NUM_WORKERS
ROW_CHUNK = 2
NUM_CHUNKS = ROWS_PER_WORKER // ROW_CHUNK


def _make_kernel():
    mesh = plsc.VectorSubcoreMesh(core_axis_name="c", subcore_axis_name="s")

    @functools.partial(
        pl.kernel,
        out_type=jax.ShapeDtypeStruct((ROWS, COLS), jnp.int32),
        mesh=mesh,
        compiler_params=pltpu.CompilerParams(needs_layout_passes=False),
        scratch_types=[
            pltpu.VMEM((ROW_CHUNK, COLS), jnp.float32),
            pltpu.VMEM((ROW_CHUNK, COLS), jnp.float32),
            pltpu.VMEM((ROW_CHUNK, COLS), jnp.int32),
            pltpu.VMEM((ROW_CHUNK, COLS), jnp.int32),
            pltpu.VMEM((72,), jnp.float32),
            pltpu.VMEM((16,), jnp.float32),
            pltpu.VMEM((16,), jnp.float32),
            pltpu.SemaphoreType.DMA,
            pltpu.SemaphoreType.DMA,
            pltpu.SemaphoreType.DMA,
            pltpu.SemaphoreType.DMA,
        ],
    )
    def bucketize(x_hbm, bpad_hbm, b0_hbm, inv_hbm, out_hbm,
                  in_v0, in_v1, out_v0, out_v1, bpad_v, b0_v, inv_v,
                  sem_i0, sem_i1, sem_o0, sem_o1):
        wid = lax.axis_index("s") * NUM_CORES + lax.axis_index("c")
        base = wid * ROWS_PER_WORKER

        pltpu.sync_copy(bpad_hbm, bpad_v)
        pltpu.sync_copy(b0_hbm, b0_v)
        pltpu.sync_copy(inv_hbm, inv_v)

        b0 = b0_v[...]
        inv = inv_v[...]

        def compute(in_v, out_v):
            for r in range(ROW_CHUNK):
                @plsc.parallel_loop(0, COLS, step=LANES, unroll=8)
                def step(i):
                    x = in_v[r, pl.ds(i, LANES)]
                    t = (x - b0) * inv
                    t = jnp.minimum(jnp.maximum(t, -1.0), 63.5)
                    tr = t.astype(jnp.int32)
                    g = tr + 1
                    lo = plsc.load_gather(bpad_v, [g])
                    out_v[r, pl.ds(i, LANES)] = jnp.where(x < lo, tr, g)

        bufs = ((in_v0, out_v0, sem_i0, sem_o0),
                (in_v1, out_v1, sem_i1, sem_o1))

        pltpu.async_copy(x_hbm.at[pl.ds(base, ROW_CHUNK), :], in_v0, sem_i0)
        pltpu.async_copy(
            x_hbm.at[pl.ds(base + ROW_CHUNK, ROW_CHUNK), :], in_v1, sem_i1
        )

        @pl.loop(0, NUM_CHUNKS, step=2)
        def pair(ch):
            for b, (in_v, out_v, sem_i, sem_o) in enumerate(bufs):
                off = base + (ch + b) * ROW_CHUNK
                pltpu.make_async_copy(
                    x_hbm.at[pl.ds(off, ROW_CHUNK), :], in_v, sem_i
                ).wait()

                @pl.when(ch > 0)
                def _wait_out():
                    pltpu.make_async_copy(
                        out_v,
                        out_hbm.at[pl.ds(off - 2 * ROW_CHUNK, ROW_CHUNK), :],
                        sem_o,
                    ).wait()

                compute(in_v, out_v)
                pltpu.async_copy(
                    out_v, out_hbm.at[pl.ds(off, ROW_CHUNK), :], sem_o
                )

                @pl.when(ch + 2 < NUM_CHUNKS)
                def _prefetch():
                    pltpu.async_copy(
                        x_hbm.at[pl.ds(off + 2 * ROW_CHUNK, ROW_CHUNK), :],
                        in_v,
                        sem_i,
                    )

        for b, (in_v, out_v, sem_i, sem_o) in enumerate(bufs):
            off = base + (NUM_CHUNKS - 2 + b) * ROW_CHUNK
            pltpu.make_async_copy(
                out_v, out_hbm.at[pl.ds(off, ROW_CHUNK), :], sem_o
            ).wait()

    return bucketize


_BUCKETIZE = _make_kernel()


@jax.jit
def kernel(inputs, bin_boundaries):
    b = bin_boundaries.astype(jnp.float32)
    b0 = b[0]
    inv = (62.0 / (b[62] - b[0])) * jnp.float32(1.0 + 1e-5)
    bpad = (
        jnp.full((72,), jnp.inf, jnp.float32)
        .at[0]
        .set(-jnp.inf)
        .at[1:64]
        .set(b)
    )
    b0_arr = jnp.full((16,), b0, jnp.float32)
    inv_arr = jnp.full((16,), inv, jnp.float32)
    return _BUCKETIZE(inputs, bpad, b0_arr, inv_arr)
